# baseline (speedup 1.0000x reference)
import functools

import jax
import jax.numpy as jnp
from jax import lax
from jax.experimental import pallas as pl
from jax.experimental.pallas import tpu as pltpu
from jax.experimental.pallas import tpu_sc as plsc

_N = 10000
_D = 128
_E = 320000
_NC = 2
_NS = 16
_EPT = _E // (_NC * _NS)
_CHUNK = 128
_NCHUNK = 80
_EPTP = _CHUNK * _NCHUNK
_NACC = 10008
_PIECE = 40
_NPIECE = _N // _PIECE
_NBUF = 2
_NGRP = _NCHUNK // _NBUF


def _agg_body(x_hbm, src_hbm, dst_hbm, out_hbm,
              acc_sh, src_v, dst_v, rows_v, stage_v, gsems, dsems, isem):
    c = lax.axis_index("c")
    s = lax.axis_index("s")
    w = c * _NS + s
    npiece_mine = (_NPIECE + _NS - 1 - s) // _NS

    idx_desc = pltpu.async_copy(src_hbm.at[w], src_v, isem)

    def _zstore(i, _):
        for j in range(_D // 16):
            stage_v[i, pl.ds(j * 16, 16)] = jnp.zeros((16,), jnp.float32)
        return 0

    lax.fori_loop(0, _PIECE, _zstore, 0)

    def _zpiece(i, _):
        row = (s + i * _NS) * _PIECE
        pltpu.sync_copy(stage_v, acc_sh.at[pl.ds(row, _PIECE)])
        return 0

    lax.fori_loop(0, npiece_mine, _zpiece, 0)
    idx_desc.wait()
    plsc.subcore_barrier()

    def _start(i, b):
        pltpu.async_copy(dst_hbm.at[w * _NCHUNK + i], dst_v[b], dsems[b])
        pltpu.async_copy(x_hbm.at[src_v.at[i]], rows_v[b], gsems[b])

    def _finish(i, b):
        pltpu.make_async_copy(dst_hbm.at[w * _NCHUNK + i], dst_v[b],
                              dsems[b]).wait()
        pltpu.make_async_copy(x_hbm.at[src_v.at[i]], rows_v[b],
                              gsems[b]).wait()
        pltpu.sync_copy(rows_v[b], acc_sh.at[dst_v[b].at[0]], add=True)

    for b in range(_NBUF):
        _start(b, b)

    def _group(g, _):
        for b in range(_NBUF):
            i = g * _NBUF + b
            _finish(i, b)
            _start(i + _NBUF, b)
        return 0

    lax.fori_loop(0, _NGRP - 1, _group, 0)
    for b in range(_NBUF):
        _finish((_NGRP - 1) * _NBUF + b, b)

    plsc.subcore_barrier()

    def _wpiece(i, _):
        row = (s + i * _NS) * _PIECE
        pltpu.sync_copy(acc_sh.at[pl.ds(row, _PIECE)], stage_v)
        pltpu.sync_copy(stage_v, out_hbm.at[pl.ds(c * _N + row, _PIECE)])
        return 0

    lax.fori_loop(0, npiece_mine, _wpiece, 0)


_agg = pl.kernel(
    _agg_body,
    out_type=jax.ShapeDtypeStruct((_NC * _N, _D), jnp.float32),
    mesh=plsc.VectorSubcoreMesh(core_axis_name="c", subcore_axis_name="s"),
    scratch_types=[
        pltpu.VMEM_SHARED((_NACC, _D), jnp.float32),
        pltpu.VMEM((_NCHUNK, _CHUNK), jnp.int32),
        [pltpu.VMEM((1, _CHUNK), jnp.int32) for _ in range(_NBUF)],
        [pltpu.VMEM((_CHUNK, _D), jnp.float32) for _ in range(_NBUF)],
        pltpu.VMEM((_PIECE, _D), jnp.float32),
        [pltpu.SemaphoreType.DMA for _ in range(_NBUF)],
        [pltpu.SemaphoreType.DMA for _ in range(_NBUF)],
        pltpu.SemaphoreType.DMA,
    ],
)


def _mm_body(relu, p0_ref, p1_ref, x_ref, wrelT_ref, wrootT_ref, b_ref, o_ref):
    agg = p0_ref[...] + p1_ref[...]
    out = jnp.dot(agg, wrelT_ref[...],
                  preferred_element_type=jnp.float32,
                  precision=lax.Precision.HIGHEST)
    out = out + jnp.dot(x_ref[...], wrootT_ref[...],
                        preferred_element_type=jnp.float32,
                        precision=lax.Precision.HIGHEST)
    out = out + b_ref[...]
    if relu:
        out = jnp.maximum(out, 0.0)
    o_ref[...] = out


def _mm(p0, p1, x, wrelT, wrootT, b2d, relu):
    blk = 1000
    return pl.pallas_call(
        functools.partial(_mm_body, relu),
        grid=(_N // blk,),
        in_specs=[
            pl.BlockSpec((blk, _D), lambda i: (i, 0)),
            pl.BlockSpec((blk, _D), lambda i: (i, 0)),
            pl.BlockSpec((blk, _D), lambda i: (i, 0)),
            pl.BlockSpec((_D, _D), lambda i: (0, 0)),
            pl.BlockSpec((_D, _D), lambda i: (0, 0)),
            pl.BlockSpec((1, _D), lambda i: (0, 0)),
        ],
        out_specs=pl.BlockSpec((blk, _D), lambda i: (i, 0)),
        out_shape=jax.ShapeDtypeStruct((_N, _D), jnp.float32),
    )(p0, p1, x, wrelT, wrootT, b2d)


def kernel(x, edge_index, W1_rel, W1_root, b1, W2_rel, W2_root, b2):
    nw = _NC * _NS
    pad = _EPTP - _EPT
    src = jnp.concatenate(
        [edge_index[0].reshape(nw, _EPT),
         jnp.zeros((nw, pad), jnp.int32)], axis=1).reshape(nw, _NCHUNK, _CHUNK)
    dst = jnp.concatenate(
        [edge_index[1].reshape(nw, _EPT),
         jnp.full((nw, pad), _N, jnp.int32)], axis=1).reshape(
             nw * _NCHUNK, 1, _CHUNK)
    p = _agg(x, src, dst)
    h = _mm(p[:_N], p[_N:], x, W1_rel.T, W1_root.T, b1.reshape(1, _D), True)
    p = _agg(h, src, dst)
    return _mm(p[:_N], p[_N:], h, W2_rel.T, W2_root.T, b2.reshape(1, _D), False)


# speedup vs baseline: 1.7560x; 1.7560x over previous
import functools

import jax
import jax.numpy as jnp
from jax import lax
from jax.experimental import pallas as pl
from jax.experimental.pallas import tpu as pltpu
from jax.experimental.pallas import tpu_sc as plsc

_N = 10000
_D = 128
_E = 320000
_NC = 2
_NS = 16
_EPT = _E // (_NC * _NS)
_CHUNK = 96
_NCHUNK = 105
_EPTP = _CHUNK * _NCHUNK
_NACC = 10008
_PIECE = 40
_NPIECE = _N // _PIECE
_NBUF = 3
_NGRP = _NCHUNK // _NBUF


def _agg_body(x_hbm, src_hbm, dst_hbm, out_hbm,
              acc_sh, src_b, dst_b, rows_v, stage_v, gsems, isems):
    c = lax.axis_index("c")
    s = lax.axis_index("s")
    w = c * _NS + s
    ebase = w * _EPTP
    npiece_mine = (_NPIECE + _NS - 1 - s) // _NS

    def idx_start(i, b):
        off = ebase + i * _CHUNK
        pltpu.async_copy(src_hbm.at[pl.ds(off, _CHUNK)], src_b[b], isems[b])
        pltpu.async_copy(dst_hbm.at[pl.ds(off, _CHUNK)], dst_b[b], isems[b])

    def idx_wait(i, b):
        off = ebase + i * _CHUNK
        pltpu.make_async_copy(src_hbm.at[pl.ds(off, _CHUNK)], src_b[b],
                              isems[b]).wait()
        pltpu.make_async_copy(dst_hbm.at[pl.ds(off, _CHUNK)], dst_b[b],
                              isems[b]).wait()

    def gather_start(b):
        pltpu.async_copy(x_hbm.at[src_b[b]], rows_v[b], gsems[b])

    def gather_wait(b):
        pltpu.make_async_copy(x_hbm.at[src_b[b]], rows_v[b], gsems[b]).wait()

    def scatter(b):
        pltpu.sync_copy(rows_v[b], acc_sh.at[dst_b[b]], add=True)

    for b in range(_NBUF):
        idx_start(b, b)

    def _zstore(i, _):
        for j in range(_D // 16):
            stage_v[i, pl.ds(j * 16, 16)] = jnp.zeros((16,), jnp.float32)
        return 0

    lax.fori_loop(0, _PIECE, _zstore, 0)

    def _zpiece(i, _):
        row = (s + i * _NS) * _PIECE
        pltpu.sync_copy(stage_v, acc_sh.at[pl.ds(row, _PIECE)])
        return 0

    lax.fori_loop(0, npiece_mine, _zpiece, 0)
    plsc.subcore_barrier()

    for b in range(_NBUF - 1):
        idx_wait(b, b)
        gather_start(b)

    def _group(g, _):
        for b in range(_NBUF):
            i = g * _NBUF + b
            gather_wait(b)
            scatter(b)
            idx_start(i + _NBUF, b)
            b2 = (b + 2) % _NBUF
            idx_wait(i + 2, b2)
            gather_start(b2)
        return 0

    lax.fori_loop(0, _NGRP - 1, _group, 0)
    gather_wait(0)
    scatter(0)
    idx_wait(_NCHUNK - 1, 2)
    gather_start(2)
    gather_wait(1)
    scatter(1)
    gather_wait(2)
    scatter(2)

    plsc.subcore_barrier()

    def _wpiece(i, _):
        row = (s + i * _NS) * _PIECE
        pltpu.sync_copy(acc_sh.at[pl.ds(row, _PIECE)], stage_v)
        pltpu.sync_copy(stage_v, out_hbm.at[pl.ds(c * _N + row, _PIECE)])
        return 0

    lax.fori_loop(0, npiece_mine, _wpiece, 0)


_agg = pl.kernel(
    _agg_body,
    out_type=jax.ShapeDtypeStruct((_NC * _N, _D), jnp.float32),
    mesh=plsc.VectorSubcoreMesh(core_axis_name="c", subcore_axis_name="s"),
    scratch_types=[
        pltpu.VMEM_SHARED((_NACC, _D), jnp.float32),
        [pltpu.VMEM((_CHUNK,), jnp.int32) for _ in range(_NBUF)],
        [pltpu.VMEM((_CHUNK,), jnp.int32) for _ in range(_NBUF)],
        [pltpu.VMEM((_CHUNK, _D), jnp.float32) for _ in range(_NBUF)],
        pltpu.VMEM((_PIECE, _D), jnp.float32),
        [pltpu.SemaphoreType.DMA for _ in range(_NBUF)],
        [pltpu.SemaphoreType.DMA for _ in range(_NBUF)],
    ],
)


def _mm_body(relu, p0_ref, p1_ref, x_ref, wrelT_ref, wrootT_ref, b_ref, o_ref):
    agg = p0_ref[...] + p1_ref[...]
    out = jnp.dot(agg, wrelT_ref[...],
                  preferred_element_type=jnp.float32,
                  precision=lax.Precision.HIGHEST)
    out = out + jnp.dot(x_ref[...], wrootT_ref[...],
                        preferred_element_type=jnp.float32,
                        precision=lax.Precision.HIGHEST)
    out = out + b_ref[...]
    if relu:
        out = jnp.maximum(out, 0.0)
    o_ref[...] = out


def _mm(p0, p1, x, wrelT, wrootT, b2d, relu):
    blk = 1000
    return pl.pallas_call(
        functools.partial(_mm_body, relu),
        grid=(_N // blk,),
        in_specs=[
            pl.BlockSpec((blk, _D), lambda i: (i, 0)),
            pl.BlockSpec((blk, _D), lambda i: (i, 0)),
            pl.BlockSpec((blk, _D), lambda i: (i, 0)),
            pl.BlockSpec((_D, _D), lambda i: (0, 0)),
            pl.BlockSpec((_D, _D), lambda i: (0, 0)),
            pl.BlockSpec((1, _D), lambda i: (0, 0)),
        ],
        out_specs=pl.BlockSpec((blk, _D), lambda i: (i, 0)),
        out_shape=jax.ShapeDtypeStruct((_N, _D), jnp.float32),
    )(p0, p1, x, wrelT, wrootT, b2d)


def kernel(x, edge_index, W1_rel, W1_root, b1, W2_rel, W2_root, b2):
    nw = _NC * _NS
    pad = _EPTP - _EPT
    src = jnp.concatenate(
        [edge_index[0].reshape(nw, _EPT),
         jnp.zeros((nw, pad), jnp.int32)], axis=1).reshape(nw * _EPTP)
    dst = jnp.concatenate(
        [edge_index[1].reshape(nw, _EPT),
         jnp.full((nw, pad), _N, jnp.int32)], axis=1).reshape(nw * _EPTP)
    p = _agg(x, src, dst)
    h = _mm(p[:_N], p[_N:], x, W1_rel.T, W1_root.T, b1.reshape(1, _D), True)
    p = _agg(h, src, dst)
    return _mm(p[:_N], p[_N:], h, W2_rel.T, W2_root.T, b2.reshape(1, _D), False)


# speedup vs baseline: 1.7829x; 1.0153x over previous
import functools

import jax
import jax.numpy as jnp
from jax import lax
from jax.experimental import pallas as pl
from jax.experimental.pallas import tpu as pltpu
from jax.experimental.pallas import tpu_sc as plsc

_N = 10000
_D = 128
_E = 320000
_NC = 2
_NS = 16
_EPT = _E // (_NC * _NS)
_CHUNK = 112
_NCHUNK = 90
_EPTP = _CHUNK * _NCHUNK
_NACC = 10008
_PIECE = 80
_NPIECE = _N // _PIECE
_NBUF = 3
_NGRP = _NCHUNK // _NBUF


def _agg_body(x_hbm, idx_hbm, out_hbm, acc_sh, pk_b, rows_v, gsems, isems):
    c = lax.axis_index("c")
    s = lax.axis_index("s")
    w = c * _NS + s
    cbase = w * _NCHUNK
    npiece_mine = (_NPIECE + _NS - 1 - s) // _NS

    def idx_start(i, b):
        pltpu.async_copy(idx_hbm.at[cbase + i], pk_b[b], isems[b])

    def idx_wait(i, b):
        pltpu.make_async_copy(idx_hbm.at[cbase + i], pk_b[b], isems[b]).wait()

    def gather_start(b):
        pltpu.async_copy(x_hbm.at[pk_b[b].at[0]], rows_v[b], gsems[b])

    def gather_wait(b):
        pltpu.make_async_copy(x_hbm.at[pk_b[b].at[0]], rows_v[b],
                              gsems[b]).wait()

    def scatter(b):
        pltpu.sync_copy(rows_v[b], acc_sh.at[pk_b[b].at[1]], add=True)

    for b in range(_NBUF):
        idx_start(b, b)

    stage = rows_v[0].at[pl.ds(0, _PIECE)]

    def _zstore(i, _):
        for j in range(_D // 16):
            rows_v[0][i, pl.ds(j * 16, 16)] = jnp.zeros((16,), jnp.float32)
        return 0

    lax.fori_loop(0, _PIECE, _zstore, 0)

    def _zpiece(i, _):
        row = (s + i * _NS) * _PIECE
        pltpu.sync_copy(stage, acc_sh.at[pl.ds(row, _PIECE)])
        return 0

    lax.fori_loop(0, npiece_mine, _zpiece, 0)
    plsc.subcore_barrier()

    for b in range(_NBUF - 1):
        idx_wait(b, b)
        gather_start(b)

    def _group(g, _):
        for b in range(_NBUF):
            i = g * _NBUF + b
            gather_wait(b)
            scatter(b)
            idx_start(i + _NBUF, b)
            b2 = (b + 2) % _NBUF
            idx_wait(i + 2, b2)
            gather_start(b2)
        return 0

    lax.fori_loop(0, _NGRP - 1, _group, 0)
    gather_wait(0)
    scatter(0)
    idx_wait(_NCHUNK - 1, 2)
    gather_start(2)
    gather_wait(1)
    scatter(1)
    gather_wait(2)
    scatter(2)

    plsc.subcore_barrier()

    def _wpiece(i, _):
        row = (s + i * _NS) * _PIECE
        pltpu.sync_copy(acc_sh.at[pl.ds(row, _PIECE)], stage)
        pltpu.sync_copy(stage, out_hbm.at[pl.ds(c * _N + row, _PIECE)])
        return 0

    lax.fori_loop(0, npiece_mine, _wpiece, 0)


_agg = pl.kernel(
    _agg_body,
    out_type=jax.ShapeDtypeStruct((_NC * _N, _D), jnp.float32),
    mesh=plsc.VectorSubcoreMesh(core_axis_name="c", subcore_axis_name="s"),
    scratch_types=[
        pltpu.VMEM_SHARED((_NACC, _D), jnp.float32),
        [pltpu.VMEM((2, _CHUNK), jnp.int32) for _ in range(_NBUF)],
        [pltpu.VMEM((_CHUNK, _D), jnp.float32) for _ in range(_NBUF)],
        [pltpu.SemaphoreType.DMA for _ in range(_NBUF)],
        [pltpu.SemaphoreType.DMA for _ in range(_NBUF)],
    ],
)


def _mm_body(relu, p0_ref, p1_ref, x_ref, wrelT_ref, wrootT_ref, b_ref, o_ref):
    agg = p0_ref[...] + p1_ref[...]
    out = jnp.dot(agg, wrelT_ref[...],
                  preferred_element_type=jnp.float32,
                  precision=lax.Precision.HIGHEST)
    out = out + jnp.dot(x_ref[...], wrootT_ref[...],
                        preferred_element_type=jnp.float32,
                        precision=lax.Precision.HIGHEST)
    out = out + b_ref[...]
    if relu:
        out = jnp.maximum(out, 0.0)
    o_ref[...] = out


def _mm(p0, p1, x, wrelT, wrootT, b2d, relu):
    blk = 1000
    return pl.pallas_call(
        functools.partial(_mm_body, relu),
        grid=(_N // blk,),
        in_specs=[
            pl.BlockSpec((blk, _D), lambda i: (i, 0)),
            pl.BlockSpec((blk, _D), lambda i: (i, 0)),
            pl.BlockSpec((blk, _D), lambda i: (i, 0)),
            pl.BlockSpec((_D, _D), lambda i: (0, 0)),
            pl.BlockSpec((_D, _D), lambda i: (0, 0)),
            pl.BlockSpec((1, _D), lambda i: (0, 0)),
        ],
        out_specs=pl.BlockSpec((blk, _D), lambda i: (i, 0)),
        out_shape=jax.ShapeDtypeStruct((_N, _D), jnp.float32),
    )(p0, p1, x, wrelT, wrootT, b2d)


def kernel(x, edge_index, W1_rel, W1_root, b1, W2_rel, W2_root, b2):
    nw = _NC * _NS
    pad = _EPTP - _EPT
    src = jnp.concatenate(
        [edge_index[0].reshape(nw, _EPT),
         jnp.zeros((nw, pad), jnp.int32)], axis=1)
    dst = jnp.concatenate(
        [edge_index[1].reshape(nw, _EPT),
         jnp.full((nw, pad), _N, jnp.int32)], axis=1)
    idx = jnp.stack(
        [src.reshape(nw, _NCHUNK, _CHUNK), dst.reshape(nw, _NCHUNK, _CHUNK)],
        axis=2).reshape(nw * _NCHUNK, 2, _CHUNK)
    p = _agg(x, idx)
    h = _mm(p[:_N], p[_N:], x, W1_rel.T, W1_root.T, b1.reshape(1, _D), True)
    p = _agg(h, idx)
    return _mm(p[:_N], p[_N:], h, W2_rel.T, W2_root.T, b2.reshape(1, _D), False)
